# baseline (speedup 1.0000x reference)
import math

import jax
import jax.numpy as jnp
from jax import lax
from jax.experimental import pallas as pl
from jax.experimental.pallas import tpu as pltpu
from jax.experimental.pallas import tpu_sc as plsc

_SINK_SIZE = 16
_SLIDING_WINDOW = 4080
_NUM_KV_HEADS = 8
_HEAD_SIZE = 128
_BLOCK_SIZE = 16
_X = 8
_NUM_BLOCKS = 2048
_BATCH = 64
_ROPE_BASE = 10000.0

_CACHE_SIZE = float(_SLIDING_WINDOW + _SINK_SIZE)
_HALF = _HEAD_SIZE // 2
_NROWS = 16384
_NC = 2
_NS = 16
_NW = _NC * _NS
_NGROUPS = _NUM_KV_HEADS * (_HEAD_SIZE // _X // 2)
_GPW = _NGROUPS // _NW
_CR = 8
_CPG = 128 // _CR
_CPW = _GPW * _CPG
_NSLOT = 2


def _tables_body(btc_ref, posc_ref, cos_ref, sin_ref):
    btc = btc_ref[...]
    posc = posc_ref[...]

    iota_b = lax.broadcasted_iota(jnp.int32, (_BATCH, _NUM_BLOCKS), 1)
    eq = btc == iota_b
    theta = jnp.maximum(posc.astype(jnp.float32) - _CACHE_SIZE, 0.0)
    masked = jnp.where(eq, jnp.broadcast_to(theta, (_BATCH, _NUM_BLOCKS)), 0.0)
    angle = jnp.sum(masked, axis=0, keepdims=True)

    fcol = lax.broadcasted_iota(jnp.int32, (_HALF, 1), 0).astype(jnp.float32)
    inv_freq = jnp.exp(fcol * (-2.0 * math.log(_ROPE_BASE) / _HEAD_SIZE))
    ang = inv_freq * angle
    cos_ref[...] = jnp.cos(ang)
    sin_ref[...] = jnp.sin(ang)


def _make_tables(interpret=False):
    return pl.pallas_call(
        _tables_body,
        out_shape=(
            jax.ShapeDtypeStruct((_HALF, _NUM_BLOCKS), jnp.float32),
            jax.ShapeDtypeStruct((_HALF, _NUM_BLOCKS), jnp.float32),
        ),
        interpret=interpret,
    )


def _sc_body(in_hbm, c_hbm, s_hbm, out_hbm,
             bufl, bufh, c_v, s_v, inl_sems, inh_sems, outl_sems, outh_sems):
    cid = lax.axis_index("c")
    sid = lax.axis_index("s")
    wid = sid * _NC + cid

    def rows_of(k):
        g = wid * _GPW + k // _CPG
        kc = k % _CPG
        h = g // 8
        dx = g - h * 8
        low = h * 2048 + dx * 128 + kc * _CR
        return low, dx, kc

    def start_in(k):
        low, _, _ = rows_of(k)
        slot = k % _NSLOT
        pltpu.make_async_copy(
            in_hbm.at[pl.ds(low, _CR)],
            bufl.at[pl.ds(slot * _CR, _CR)],
            inl_sems.at[slot]).start()
        pltpu.make_async_copy(
            in_hbm.at[pl.ds(low + 1024, _CR)],
            bufh.at[pl.ds(slot * _CR, _CR)],
            inh_sems.at[slot]).start()

    def wait_in(k):
        low, _, _ = rows_of(k)
        slot = k % _NSLOT
        pltpu.make_async_copy(
            in_hbm.at[pl.ds(low, _CR)],
            bufl.at[pl.ds(slot * _CR, _CR)],
            inl_sems.at[slot]).wait()
        pltpu.make_async_copy(
            in_hbm.at[pl.ds(low + 1024, _CR)],
            bufh.at[pl.ds(slot * _CR, _CR)],
            inh_sems.at[slot]).wait()

    def start_out(k):
        low, _, _ = rows_of(k)
        slot = k % _NSLOT
        pltpu.make_async_copy(
            bufl.at[pl.ds(slot * _CR, _CR)],
            out_hbm.at[pl.ds(low, _CR)],
            outl_sems.at[slot]).start()
        pltpu.make_async_copy(
            bufh.at[pl.ds(slot * _CR, _CR)],
            out_hbm.at[pl.ds(low + 1024, _CR)],
            outh_sems.at[slot]).start()

    def wait_out(k):
        low, _, _ = rows_of(k)
        slot = k % _NSLOT
        pltpu.make_async_copy(
            bufl.at[pl.ds(slot * _CR, _CR)],
            out_hbm.at[pl.ds(low, _CR)],
            outl_sems.at[slot]).wait()
        pltpu.make_async_copy(
            bufh.at[pl.ds(slot * _CR, _CR)],
            out_hbm.at[pl.ds(low + 1024, _CR)],
            outh_sems.at[slot]).wait()

    start_in(0)
    start_in(1)

    def step(k, carry):
        _, dx, kc = rows_of(k)
        slot = k % _NSLOT

        @pl.when(kc == 0)
        def _():
            pltpu.sync_copy(c_hbm.at[pl.ds(dx * 8, 8)], c_v)
            pltpu.sync_copy(s_hbm.at[pl.ds(dx * 8, 8)], s_v)

        wait_in(k)

        def comp(v4, carry2):
            for u in range(4):
                o = (v4 * 4 + u) * 16
                for i in range(_CR):
                    c = c_v[i, pl.ds(o, 16)]
                    s = s_v[i, pl.ds(o, 16)]
                    k1 = bufl[slot * _CR + i, pl.ds(o, 16)]
                    k2 = bufh[slot * _CR + i, pl.ds(o, 16)]
                    bufl[slot * _CR + i, pl.ds(o, 16)] = k1 * c - k2 * s
                    bufh[slot * _CR + i, pl.ds(o, 16)] = k2 * c + k1 * s
            return carry2

        lax.fori_loop(0, _NUM_BLOCKS // 64, comp, 0)
        start_out(k)

        @pl.when(k + 2 < _CPW)
        def _():
            wait_out(k)
            start_in(k + 2)

        @pl.when(k + 2 >= _CPW)
        def _():
            wait_out(k)

        return carry

    lax.fori_loop(0, _CPW, step, 0)


def _make_sc_apply(interpret=False):
    mesh = plsc.VectorSubcoreMesh(
        core_axis_name="c", subcore_axis_name="s",
        num_cores=_NC, num_subcores=_NS)
    return pl.kernel(
        _sc_body,
        out_type=jax.ShapeDtypeStruct((_NROWS, _NUM_BLOCKS), jnp.float32),
        mesh=mesh,
        compiler_params=pltpu.CompilerParams(
            needs_layout_passes=False, use_tc_tiling_on_sc=True),
        scratch_types=[
            pltpu.VMEM((_NSLOT * _CR, _NUM_BLOCKS), jnp.float32),
            pltpu.VMEM((_NSLOT * _CR, _NUM_BLOCKS), jnp.float32),
            pltpu.VMEM((8, _NUM_BLOCKS), jnp.float32),
            pltpu.VMEM((8, _NUM_BLOCKS), jnp.float32),
            pltpu.SemaphoreType.DMA((_NSLOT,)),
            pltpu.SemaphoreType.DMA((_NSLOT,)),
            pltpu.SemaphoreType.DMA((_NSLOT,)),
            pltpu.SemaphoreType.DMA((_NSLOT,)),
        ],
        interpret=interpret,
    )


def _kernel_impl(key_cache, block_tables, context_lens, positions,
                 interpret=False):
    del context_lens
    m = jnp.transpose(key_cache, (1, 2, 3, 4, 0)).reshape(_NROWS, _NUM_BLOCKS)
    btc = block_tables[:, :1]
    posc = positions.reshape(_BATCH, 1)
    cos_t, sin_t = _make_tables(interpret)(btc, posc)
    out = _make_sc_apply(interpret)(m, cos_t, sin_t)
    out5 = out.reshape(_NUM_KV_HEADS, _HEAD_SIZE // _X, _BLOCK_SIZE, _X,
                       _NUM_BLOCKS)
    return jnp.transpose(out5, (4, 0, 1, 2, 3))


def kernel(key_cache, block_tables, context_lens, positions):
    return _kernel_impl(key_cache, block_tables, context_lens, positions)


# speedup vs baseline: 1.1321x; 1.1321x over previous
import math

import jax
import jax.numpy as jnp
from jax import lax
from jax.experimental import pallas as pl
from jax.experimental.pallas import tpu as pltpu
from jax.experimental.pallas import tpu_sc as plsc

_SINK_SIZE = 16
_SLIDING_WINDOW = 4080
_NUM_KV_HEADS = 8
_HEAD_SIZE = 128
_BLOCK_SIZE = 16
_X = 8
_NUM_BLOCKS = 2048
_BATCH = 64
_ROPE_BASE = 10000.0

_CACHE_SIZE = float(_SLIDING_WINDOW + _SINK_SIZE)
_HALF = _HEAD_SIZE // 2
_NROWS = 16384
_NC = 2
_NS = 16
_NW = _NC * _NS
_NGROUPS = _NUM_KV_HEADS * (_HEAD_SIZE // _X // 2)
_GPW = _NGROUPS // _NW
_CR = 8
_CW = 1024
_CPG = (128 // _CR) * 2
_CPW = _GPW * _CPG
_NSLOT = 3


def _tables_body(btc_ref, posc_ref, cos_ref, sin_ref):
    btc = btc_ref[...]
    posc = posc_ref[...]

    iota_b = lax.broadcasted_iota(jnp.int32, (_BATCH, _NUM_BLOCKS), 1)
    eq = btc == iota_b
    theta = jnp.maximum(posc.astype(jnp.float32) - _CACHE_SIZE, 0.0)
    masked = jnp.where(eq, jnp.broadcast_to(theta, (_BATCH, _NUM_BLOCKS)), 0.0)
    angle = jnp.sum(masked, axis=0, keepdims=True)

    fcol = lax.broadcasted_iota(jnp.int32, (_HALF, 1), 0).astype(jnp.float32)
    inv_freq = jnp.exp(fcol * (-2.0 * math.log(_ROPE_BASE) / _HEAD_SIZE))
    ang = inv_freq * angle
    cos_ref[...] = jnp.cos(ang)
    sin_ref[...] = jnp.sin(ang)


def _make_tables(interpret=False):
    return pl.pallas_call(
        _tables_body,
        out_shape=(
            jax.ShapeDtypeStruct((_HALF, _NUM_BLOCKS), jnp.float32),
            jax.ShapeDtypeStruct((_HALF, _NUM_BLOCKS), jnp.float32),
        ),
        interpret=interpret,
    )


def _sc_body(in_hbm, c_hbm, s_hbm, out_hbm,
             bufl, bufh, c_v, s_v, inl_sems, inh_sems, outl_sems, outh_sems):
    cid = lax.axis_index("c")
    sid = lax.axis_index("s")
    wid = sid * _NC + cid

    def rows_of(k):
        g = wid * _GPW + k // _CPG
        rem = k % _CPG
        lh = rem // 16
        t = rem - lh * 16
        h = g // 8
        dx = g - h * 8
        low = h * 2048 + dx * 128 + t * _CR
        return low, lh * _CW, dx, t

    def start_in(k):
        low, lb, _, _ = rows_of(k)
        slot = k % _NSLOT
        pltpu.make_async_copy(
            in_hbm.at[pl.ds(low, _CR), pl.ds(lb, _CW)],
            bufl.at[pl.ds(slot * _CR, _CR)],
            inl_sems.at[slot]).start()
        pltpu.make_async_copy(
            in_hbm.at[pl.ds(low + 1024, _CR), pl.ds(lb, _CW)],
            bufh.at[pl.ds(slot * _CR, _CR)],
            inh_sems.at[slot]).start()

    def wait_in(k):
        low, lb, _, _ = rows_of(k)
        slot = k % _NSLOT
        pltpu.make_async_copy(
            in_hbm.at[pl.ds(low, _CR), pl.ds(lb, _CW)],
            bufl.at[pl.ds(slot * _CR, _CR)],
            inl_sems.at[slot]).wait()
        pltpu.make_async_copy(
            in_hbm.at[pl.ds(low + 1024, _CR), pl.ds(lb, _CW)],
            bufh.at[pl.ds(slot * _CR, _CR)],
            inh_sems.at[slot]).wait()

    def start_out(k):
        low, lb, _, _ = rows_of(k)
        slot = k % _NSLOT
        pltpu.make_async_copy(
            bufl.at[pl.ds(slot * _CR, _CR)],
            out_hbm.at[pl.ds(low, _CR), pl.ds(lb, _CW)],
            outl_sems.at[slot]).start()
        pltpu.make_async_copy(
            bufh.at[pl.ds(slot * _CR, _CR)],
            out_hbm.at[pl.ds(low + 1024, _CR), pl.ds(lb, _CW)],
            outh_sems.at[slot]).start()

    def wait_out(k):
        low, lb, _, _ = rows_of(k)
        slot = k % _NSLOT
        pltpu.make_async_copy(
            bufl.at[pl.ds(slot * _CR, _CR)],
            out_hbm.at[pl.ds(low, _CR), pl.ds(lb, _CW)],
            outl_sems.at[slot]).wait()
        pltpu.make_async_copy(
            bufh.at[pl.ds(slot * _CR, _CR)],
            out_hbm.at[pl.ds(low + 1024, _CR), pl.ds(lb, _CW)],
            outh_sems.at[slot]).wait()

    start_in(0)
    start_in(1)

    def step(k, carry):
        _, lb, dx, t = rows_of(k)
        slot = k % _NSLOT

        @pl.when(t == 0)
        def _():
            pltpu.sync_copy(c_hbm.at[pl.ds(dx * 8, 8), pl.ds(lb, _CW)], c_v)
            pltpu.sync_copy(s_hbm.at[pl.ds(dx * 8, 8), pl.ds(lb, _CW)], s_v)

        wait_in(k)

        def comp(v4, carry2):
            for u in range(4):
                o = (v4 * 4 + u) * 16
                for i in range(_CR):
                    c = c_v[i, pl.ds(o, 16)]
                    s = s_v[i, pl.ds(o, 16)]
                    k1 = bufl[slot * _CR + i, pl.ds(o, 16)]
                    k2 = bufh[slot * _CR + i, pl.ds(o, 16)]
                    bufl[slot * _CR + i, pl.ds(o, 16)] = k1 * c - k2 * s
                    bufh[slot * _CR + i, pl.ds(o, 16)] = k2 * c + k1 * s
            return carry2

        lax.fori_loop(0, _CW // 64, comp, 0)
        start_out(k)

        @pl.when(k >= 1)
        def _():
            wait_out(k - 1)

        @pl.when(k + 2 < _CPW)
        def _():
            start_in(k + 2)

        return carry

    lax.fori_loop(0, _CPW, step, 0)
    wait_out(_CPW - 1)


def _make_sc_apply(interpret=False):
    mesh = plsc.VectorSubcoreMesh(
        core_axis_name="c", subcore_axis_name="s",
        num_cores=_NC, num_subcores=_NS)
    return pl.kernel(
        _sc_body,
        out_type=jax.ShapeDtypeStruct((_NROWS, _NUM_BLOCKS), jnp.float32),
        mesh=mesh,
        compiler_params=pltpu.CompilerParams(
            needs_layout_passes=False, use_tc_tiling_on_sc=True),
        scratch_types=[
            pltpu.VMEM((_NSLOT * _CR, _CW), jnp.float32),
            pltpu.VMEM((_NSLOT * _CR, _CW), jnp.float32),
            pltpu.VMEM((8, _CW), jnp.float32),
            pltpu.VMEM((8, _CW), jnp.float32),
            pltpu.SemaphoreType.DMA((_NSLOT,)),
            pltpu.SemaphoreType.DMA((_NSLOT,)),
            pltpu.SemaphoreType.DMA((_NSLOT,)),
            pltpu.SemaphoreType.DMA((_NSLOT,)),
        ],
        interpret=interpret,
    )


def _kernel_impl(key_cache, block_tables, context_lens, positions,
                 interpret=False):
    del context_lens
    m = jnp.transpose(key_cache, (1, 2, 3, 4, 0)).reshape(_NROWS, _NUM_BLOCKS)
    btc = block_tables[:, :1]
    posc = positions.reshape(_BATCH, 1)
    cos_t, sin_t = _make_tables(interpret)(btc, posc)
    out = _make_sc_apply(interpret)(m, cos_t, sin_t)
    out5 = out.reshape(_NUM_KV_HEADS, _HEAD_SIZE // _X, _BLOCK_SIZE, _X,
                       _NUM_BLOCKS)
    return jnp.transpose(out5, (4, 0, 1, 2, 3))


def kernel(key_cache, block_tables, context_lens, positions):
    return _kernel_impl(key_cache, block_tables, context_lens, positions)


# speedup vs baseline: 1.2201x; 1.0778x over previous
import math

import jax
import jax.numpy as jnp
from jax import lax
from jax.experimental import pallas as pl
from jax.experimental.pallas import tpu as pltpu
from jax.experimental.pallas import tpu_sc as plsc

_SINK_SIZE = 16
_SLIDING_WINDOW = 4080
_NUM_KV_HEADS = 8
_HEAD_SIZE = 128
_BLOCK_SIZE = 16
_X = 8
_NUM_BLOCKS = 2048
_BATCH = 64
_ROPE_BASE = 10000.0

_CACHE_SIZE = float(_SLIDING_WINDOW + _SINK_SIZE)
_HALF = _HEAD_SIZE // 2
_NROWS = 16384
_NC = 2
_NS = 16
_NW = _NC * _NS
_NGROUPS = _NUM_KV_HEADS * (_HEAD_SIZE // _X // 2)
_GPW = _NGROUPS // _NW
_NTR = 2048
_NTC = 16
_TCH = 8
_CPW = 64
_NSLOT = 3


def _tables_body(btc_ref, posc_ref, cos_ref, sin_ref):
    btc = btc_ref[...]
    posc = posc_ref[...]

    iota_b = lax.broadcasted_iota(jnp.int32, (_BATCH, _NUM_BLOCKS), 1)
    eq = btc == iota_b
    theta = jnp.maximum(posc.astype(jnp.float32) - _CACHE_SIZE, 0.0)
    masked = jnp.where(eq, jnp.broadcast_to(theta, (_BATCH, _NUM_BLOCKS)), 0.0)
    angle = jnp.sum(masked, axis=0, keepdims=True)

    fcol = lax.broadcasted_iota(jnp.int32, (_HALF, 1), 0).astype(jnp.float32)
    inv_freq = jnp.exp(fcol * (-2.0 * math.log(_ROPE_BASE) / _HEAD_SIZE))
    ang = inv_freq * angle
    cos_ref[...] = jnp.cos(ang)
    sin_ref[...] = jnp.sin(ang)


def _make_tables(interpret=False):
    return pl.pallas_call(
        _tables_body,
        out_shape=(
            jax.ShapeDtypeStruct((_HALF, _NUM_BLOCKS), jnp.float32),
            jax.ShapeDtypeStruct((_HALF, _NUM_BLOCKS), jnp.float32),
        ),
        interpret=interpret,
    )


def _sc_body(in_hbm, c_hbm, s_hbm, out_hbm,
             bufl, bufh, c_v, s_v, inl_sems, inh_sems, outl_sems, outh_sems):
    cid = lax.axis_index("c")
    sid = lax.axis_index("s")
    wid = sid * _NC + cid
    dx = wid % 8
    hpair = wid // 8

    def loc_of(k):
        tch = k // 32
        rem = k - tch * 32
        h = hpair * 2 + rem // 16
        t = rem % 16
        low = h * 256 + dx * 16 + t
        return low, tch * _TCH, rem

    def start_in(k):
        low, tcb, _ = loc_of(k)
        slot = k % _NSLOT
        pltpu.make_async_copy(
            in_hbm.at[low, pl.ds(tcb, _TCH)],
            bufl.at[slot], inl_sems.at[slot]).start()
        pltpu.make_async_copy(
            in_hbm.at[low + 128, pl.ds(tcb, _TCH)],
            bufh.at[slot], inh_sems.at[slot]).start()

    def wait_in(k):
        low, tcb, _ = loc_of(k)
        slot = k % _NSLOT
        pltpu.make_async_copy(
            in_hbm.at[low, pl.ds(tcb, _TCH)],
            bufl.at[slot], inl_sems.at[slot]).wait()
        pltpu.make_async_copy(
            in_hbm.at[low + 128, pl.ds(tcb, _TCH)],
            bufh.at[slot], inh_sems.at[slot]).wait()

    def start_out(k):
        low, tcb, _ = loc_of(k)
        slot = k % _NSLOT
        pltpu.make_async_copy(
            bufl.at[slot], out_hbm.at[low, pl.ds(tcb, _TCH)],
            outl_sems.at[slot]).start()
        pltpu.make_async_copy(
            bufh.at[slot], out_hbm.at[low + 128, pl.ds(tcb, _TCH)],
            outh_sems.at[slot]).start()

    def wait_out(k):
        low, tcb, _ = loc_of(k)
        slot = k % _NSLOT
        pltpu.make_async_copy(
            bufl.at[slot], out_hbm.at[low, pl.ds(tcb, _TCH)],
            outl_sems.at[slot]).wait()
        pltpu.make_async_copy(
            bufh.at[slot], out_hbm.at[low + 128, pl.ds(tcb, _TCH)],
            outh_sems.at[slot]).wait()

    start_in(0)
    start_in(1)

    def step(k, carry):
        _, tcb, rem = loc_of(k)
        slot = k % _NSLOT

        @pl.when(rem == 0)
        def _():
            pltpu.sync_copy(
                c_hbm.at[pl.ds(dx * 8, 8), pl.ds(tcb, _TCH)], c_v)
            pltpu.sync_copy(
                s_hbm.at[pl.ds(dx * 8, 8), pl.ds(tcb, _TCH)], s_v)

        wait_in(k)

        def comp(m, carry2):
            tc = m // 8
            o = (m - tc * 8) * 16
            for x in range(8):
                c = c_v[x, tc, pl.ds(o, 16)]
                s = s_v[x, tc, pl.ds(o, 16)]
                k1 = bufl[slot, tc, x, pl.ds(o, 16)]
                k2 = bufh[slot, tc, x, pl.ds(o, 16)]
                bufl[slot, tc, x, pl.ds(o, 16)] = k1 * c - k2 * s
                bufh[slot, tc, x, pl.ds(o, 16)] = k2 * c + k1 * s
            return carry2

        lax.fori_loop(0, _TCH * 8, comp, 0)
        start_out(k)

        @pl.when(k >= 1)
        def _():
            wait_out(k - 1)

        @pl.when(k + 2 < _CPW)
        def _():
            start_in(k + 2)

        return carry

    lax.fori_loop(0, _CPW, step, 0)
    wait_out(_CPW - 1)


def _make_sc_apply(interpret=False):
    mesh = plsc.VectorSubcoreMesh(
        core_axis_name="c", subcore_axis_name="s",
        num_cores=_NC, num_subcores=_NS)
    return pl.kernel(
        _sc_body,
        out_type=jax.ShapeDtypeStruct((_NTR, _NTC, 8, 128), jnp.float32),
        mesh=mesh,
        compiler_params=pltpu.CompilerParams(needs_layout_passes=False),
        scratch_types=[
            pltpu.VMEM((_NSLOT, _TCH, 8, 128), jnp.float32),
            pltpu.VMEM((_NSLOT, _TCH, 8, 128), jnp.float32),
            pltpu.VMEM((8, _TCH, 128), jnp.float32),
            pltpu.VMEM((8, _TCH, 128), jnp.float32),
            pltpu.SemaphoreType.DMA((_NSLOT,)),
            pltpu.SemaphoreType.DMA((_NSLOT,)),
            pltpu.SemaphoreType.DMA((_NSLOT,)),
            pltpu.SemaphoreType.DMA((_NSLOT,)),
        ],
        interpret=interpret,
    )


def _kernel_impl(key_cache, block_tables, context_lens, positions,
                 interpret=False):
    del context_lens
    a6 = key_cache.reshape(_NTC, 128, _NUM_KV_HEADS, _HEAD_SIZE // _X,
                           _BLOCK_SIZE, _X)
    m = jnp.transpose(a6, (2, 3, 4, 0, 5, 1)).reshape(_NTR, _NTC, 8, 128)
    btc = block_tables[:, :1]
    posc = positions.reshape(_BATCH, 1)
    cos_t, sin_t = _make_tables(interpret)(btc, posc)
    c3 = cos_t.reshape(_HALF, _NTC, 128)
    s3 = sin_t.reshape(_HALF, _NTC, 128)
    out = _make_sc_apply(interpret)(m, c3, s3)
    out6 = out.reshape(_NUM_KV_HEADS, _HEAD_SIZE // _X, _BLOCK_SIZE, _NTC,
                       _X, 128)
    return jnp.transpose(out6, (3, 5, 0, 1, 2, 4)).reshape(
        _NUM_BLOCKS, _NUM_KV_HEADS, _HEAD_SIZE // _X, _BLOCK_SIZE, _X)


def kernel(key_cache, block_tables, context_lens, positions):
    return _kernel_impl(key_cache, block_tables, context_lens, positions)


# speedup vs baseline: 1.2826x; 1.0512x over previous
import functools
import math

import jax
import jax.numpy as jnp
from jax import lax
from jax.experimental import pallas as pl
from jax.experimental.pallas import tpu as pltpu
from jax.experimental.pallas import tpu_sc as plsc

_SINK_SIZE = 16
_SLIDING_WINDOW = 4080
_NUM_KV_HEADS = 8
_HEAD_SIZE = 128
_BLOCK_SIZE = 16
_X = 8
_NUM_BLOCKS = 2048
_BATCH = 64
_ROPE_BASE = 10000.0

_CACHE_SIZE = float(_SLIDING_WINDOW + _SINK_SIZE)
_ROW = _NUM_KV_HEADS * (_HEAD_SIZE // _X) * _BLOCK_SIZE * _X
_HALF = _HEAD_SIZE // 2
_NC = 2
_NS = 16
_NW = _NC * _NS
_SLOTS_PER_W = _BATCH // _NW


def _tables_body(btc_ref, btr_ref, posr_ref, cos_ref, sin_ref, enc_ref):
    btc = btc_ref[...]
    btr = btr_ref[...]
    posr = posr_ref[...]

    eq = btc == btr
    jidx = lax.broadcasted_iota(jnp.int32, (_BATCH, _BATCH), 1)
    firstj = jnp.min(jnp.where(eq, jidx, _BATCH), axis=1, keepdims=True)
    iidx = lax.broadcasted_iota(jnp.int32, (_BATCH, 1), 0)
    is_first = firstj == iidx

    theta = jnp.maximum(posr.astype(jnp.float32) - _CACHE_SIZE, 0.0)
    angle = jnp.sum(
        jnp.where(eq, jnp.broadcast_to(theta, (_BATCH, _BATCH)), 0.0),
        axis=1, keepdims=True)

    lane = lax.broadcasted_iota(jnp.int32, (_BATCH, _HEAD_SIZE), 1)
    f = (lane // 16) * 8 + (lane % 16) % 8
    inv_freq = jnp.exp(
        f.astype(jnp.float32) * (-2.0 * math.log(_ROPE_BASE) / _HEAD_SIZE))
    ang = angle * inv_freq
    cos_ref[...] = jnp.cos(ang)
    sin_ref[...] = jnp.sin(ang)

    proc = jnp.logical_and(is_first, angle > 0.0)
    enc = jnp.where(proc, btc, -1)
    enc_ref[...] = jnp.broadcast_to(enc, (_BATCH, 16))


def _make_tables(interpret=False):
    return pl.pallas_call(
        _tables_body,
        out_shape=(
            jax.ShapeDtypeStruct((_BATCH, _HEAD_SIZE), jnp.float32),
            jax.ShapeDtypeStruct((_BATCH, _HEAD_SIZE), jnp.float32),
            jax.ShapeDtypeStruct((_BATCH, 16), jnp.int32),
        ),
        interpret=interpret,
    )


def _sc_body(cache_ref, cos_hbm, sin_hbm, enc_hbm,
             row_v, cos_v, sin_v, enc_v, sem):
    cid = lax.axis_index("c")
    sid = lax.axis_index("s")
    wid = sid * _NC + cid
    for k in range(_SLOTS_PER_W):
        slot = wid * _SLOTS_PER_W + k
        pltpu.sync_copy(enc_hbm.at[pl.ds(slot, 1)], enc_v)
        blk = jnp.max(enc_v[0, :])

        @pl.when(blk >= 0)
        def _():
            pltpu.async_copy(cache_ref.at[pl.ds(blk, 1)], row_v, sem).wait()
            pltpu.sync_copy(cos_hbm.at[pl.ds(slot, 1)], cos_v)
            pltpu.sync_copy(sin_hbm.at[pl.ds(slot, 1)], sin_v)

            def body(hd, carry):
                h = hd // _X
                dx = hd - h * _X
                cbase = dx * 16
                off1 = (h * 16 + dx) * 128
                off2 = off1 + 1024
                c = cos_v[0, pl.ds(cbase, 16)]
                s = sin_v[0, pl.ds(cbase, 16)]
                for v in range(8):
                    o1 = off1 + v * 16
                    o2 = off2 + v * 16
                    k1 = row_v[0, pl.ds(o1, 16)]
                    k2 = row_v[0, pl.ds(o2, 16)]
                    row_v[0, pl.ds(o1, 16)] = k1 * c - k2 * s
                    row_v[0, pl.ds(o2, 16)] = k2 * c + k1 * s
                return carry

            lax.fori_loop(0, _NUM_KV_HEADS * _X, body, 0)
            pltpu.async_copy(row_v, cache_ref.at[pl.ds(blk, 1)], sem).wait()


def _make_sc_apply(interpret=False):
    mesh = plsc.VectorSubcoreMesh(
        core_axis_name="c", subcore_axis_name="s",
        num_cores=_NC, num_subcores=_NS)
    return pl.kernel(
        _sc_body,
        out_type=(),
        mesh=mesh,
        compiler_params=pltpu.CompilerParams(needs_layout_passes=False),
        scratch_types=[
            pltpu.VMEM((1, _ROW), jnp.float32),
            pltpu.VMEM((1, _HEAD_SIZE), jnp.float32),
            pltpu.VMEM((1, _HEAD_SIZE), jnp.float32),
            pltpu.VMEM((1, 16), jnp.int32),
            pltpu.SemaphoreType.DMA,
        ],
        interpret=interpret,
    )


def _kernel_impl(key_cache, block_tables, context_lens, positions,
                 interpret=False):
    del context_lens
    shape = key_cache.shape
    cache2 = key_cache.reshape(_NUM_BLOCKS, _ROW)
    btc = block_tables[:, :1]
    btr = btc.reshape(1, _BATCH)
    posr = positions.reshape(1, _BATCH)
    cos_t, sin_t, enc = _make_tables(interpret)(btc, btr, posr)
    cache_ref = jax.new_ref(cache2)
    _make_sc_apply(interpret)(cache_ref, cos_t, sin_t, enc)
    return cache_ref[...].reshape(shape)


def kernel(key_cache, block_tables, context_lens, positions):
    return _kernel_impl(key_cache, block_tables, context_lens, positions)
